# baseline (speedup 1.0000x reference)
import jax
import jax.numpy as jnp
from jax.experimental import pallas as pl
from jax.experimental.pallas import tpu as pltpu

_B, _T, _DIN, _H, _C = 1024, 85, 20, 1024, 27
_OBJ = _T * 36


def _base_kernel(o_ref, oW1_ref, ob1_ref, oW2_ref, ob2_ref, eW1h_ref,
                 eb1_ref, base_ref):
    h = jnp.dot(o_ref[...], oW1_ref[...],
                preferred_element_type=jnp.float32) + ob1_ref[...]
    h = jnp.maximum(h, 0.0).astype(jnp.bfloat16)
    obj = jnp.dot(h, oW2_ref[...],
                  preferred_element_type=jnp.float32) + ob2_ref[...]
    base_ref[...] = jnp.dot(obj.astype(jnp.bfloat16), eW1h_ref[...],
                            preferred_element_type=jnp.float32) + eb1_ref[...]


def _loop_kernel(x_ref, base_ref, w1_ref, w2_ref, eb2_ref, out_ref,
                 prev_ref):
    t = pl.program_id(1)
    xt = x_ref[:, 0, 0, :]

    @pl.when(t == 0)
    def _():
        prev_ref[...] = xt[:, 0:1]

    cur = jnp.concatenate([prev_ref[...], xt[:, 1:]], axis=1)
    h = jnp.dot(cur, w1_ref[...],
                preferred_element_type=jnp.float32) + base_ref[...]
    h = jnp.maximum(h, 0.0).astype(jnp.bfloat16)
    ew = jnp.dot(h, w2_ref[...],
                 preferred_element_type=jnp.float32) + eb2_ref[...]
    prev_ref[...] = ew[:, 0:1].astype(jnp.bfloat16)
    out_ref[:, 0, 0, :] = ew


def kernel(o, x, oW1, ob1, oW2, ob2, eW1, eb1, eW2, eb2):
    bf = jnp.bfloat16
    MB = 256
    base = pl.pallas_call(
        _base_kernel,
        grid=(_B // MB,),
        in_specs=[
            pl.BlockSpec((MB, _OBJ), lambda i: (i, 0)),
            pl.BlockSpec((_OBJ, _H), lambda i: (0, 0)),
            pl.BlockSpec((1, _H), lambda i: (0, 0)),
            pl.BlockSpec((_H, _H), lambda i: (0, 0)),
            pl.BlockSpec((1, _H), lambda i: (0, 0)),
            pl.BlockSpec((_H, _H), lambda i: (0, 0)),
            pl.BlockSpec((1, _H), lambda i: (0, 0)),
        ],
        out_specs=pl.BlockSpec((MB, _H), lambda i: (i, 0)),
        out_shape=jax.ShapeDtypeStruct((_B, _H), jnp.float32),
        compiler_params=pltpu.CompilerParams(
            dimension_semantics=("parallel",),
            vmem_limit_bytes=56 * 1024 * 1024,
        ),
        name="walk_base",
    )(o.astype(bf), oW1.astype(bf), ob1.reshape(1, -1), oW2.astype(bf),
      ob2.reshape(1, -1), eW1[_DIN:].astype(bf), eb1.reshape(1, -1))

    x4 = x.astype(bf).reshape(_B, _T, 1, _DIN)
    BB = 512
    out = pl.pallas_call(
        _loop_kernel,
        grid=(_B // BB, _T - 1),
        in_specs=[
            pl.BlockSpec((BB, 1, 1, _DIN), lambda b, t: (b, t, 0, 0)),
            pl.BlockSpec((BB, _H), lambda b, t: (b, 0)),
            pl.BlockSpec((_DIN, _H), lambda b, t: (0, 0)),
            pl.BlockSpec((_H, _C), lambda b, t: (0, 0)),
            pl.BlockSpec((1, _C), lambda b, t: (0, 0)),
        ],
        out_specs=pl.BlockSpec((BB, 1, 1, _C), lambda b, t: (b, t, 0, 0)),
        out_shape=jax.ShapeDtypeStruct((_B, _T - 1, 1, _C), jnp.float32),
        scratch_shapes=[pltpu.VMEM((BB, 1), bf)],
        compiler_params=pltpu.CompilerParams(
            dimension_semantics=("parallel", "arbitrary"),
        ),
        name="walk_loop",
    )(x4, base, eW1[:_DIN].astype(bf), eW2.astype(bf), eb2.reshape(1, -1))
    return out.reshape(_B, _T - 1, _C)


# speedup vs baseline: 1.4443x; 1.4443x over previous
import jax
import jax.numpy as jnp
from jax.experimental import pallas as pl
from jax.experimental.pallas import tpu as pltpu

_B, _T, _DIN, _H, _C = 1024, 85, 20, 1024, 27
_OBJ = _T * 36


def _base_kernel(o_ref, oW1_ref, ob1_ref, oW2_ref, ob2_ref, eW1h_ref,
                 eb1_ref, base_ref):
    h = jnp.dot(o_ref[...], oW1_ref[...],
                preferred_element_type=jnp.float32) + ob1_ref[...]
    h = jnp.maximum(h, 0.0).astype(jnp.bfloat16)
    obj = jnp.dot(h, oW2_ref[...],
                  preferred_element_type=jnp.float32) + ob2_ref[...]
    base_ref[...] = jnp.dot(obj.astype(jnp.bfloat16), eW1h_ref[...],
                            preferred_element_type=jnp.float32) + eb1_ref[...]


def _loop_kernel(xT_ref, base_ref, w1_ref, w2_ref, eb2_ref, out_ref,
                 prev_ref):
    t = pl.program_id(1)
    xt = xT_ref[0]

    @pl.when(t == 0)
    def _():
        prev_ref[...] = xt[:, 0:1]

    cur = jnp.concatenate([prev_ref[...], xt[:, 1:]], axis=1)
    h = jnp.dot(cur, w1_ref[...],
                preferred_element_type=jnp.float32) + base_ref[...]
    h = jnp.maximum(h, 0.0).astype(jnp.bfloat16)
    ew = jnp.dot(h, w2_ref[...],
                 preferred_element_type=jnp.float32) + eb2_ref[...]
    prev_ref[...] = ew[:, 0:1].astype(jnp.bfloat16)
    out_ref[0] = ew


def kernel(o, x, oW1, ob1, oW2, ob2, eW1, eb1, eW2, eb2):
    bf = jnp.bfloat16
    MB = 256
    base = pl.pallas_call(
        _base_kernel,
        grid=(_B // MB,),
        in_specs=[
            pl.BlockSpec((MB, _OBJ), lambda i: (i, 0)),
            pl.BlockSpec((_OBJ, _H), lambda i: (0, 0)),
            pl.BlockSpec((1, _H), lambda i: (0, 0)),
            pl.BlockSpec((_H, _H), lambda i: (0, 0)),
            pl.BlockSpec((1, _H), lambda i: (0, 0)),
            pl.BlockSpec((_H, _H), lambda i: (0, 0)),
            pl.BlockSpec((1, _H), lambda i: (0, 0)),
        ],
        out_specs=pl.BlockSpec((MB, _H), lambda i: (i, 0)),
        out_shape=jax.ShapeDtypeStruct((_B, _H), jnp.float32),
        compiler_params=pltpu.CompilerParams(
            dimension_semantics=("parallel",),
            vmem_limit_bytes=56 * 1024 * 1024,
        ),
        name="walk_base",
    )(o.astype(bf), oW1.astype(bf), ob1.reshape(1, -1), oW2.astype(bf),
      ob2.reshape(1, -1), eW1[_DIN:].astype(bf), eb1.reshape(1, -1))

    xT = jnp.swapaxes(x, 0, 1).astype(bf)
    BB = 512
    outT = pl.pallas_call(
        _loop_kernel,
        grid=(_B // BB, _T - 1),
        in_specs=[
            pl.BlockSpec((1, BB, _DIN), lambda b, t: (t, b, 0)),
            pl.BlockSpec((BB, _H), lambda b, t: (b, 0)),
            pl.BlockSpec((_DIN, _H), lambda b, t: (0, 0)),
            pl.BlockSpec((_H, _C), lambda b, t: (0, 0)),
            pl.BlockSpec((1, _C), lambda b, t: (0, 0)),
        ],
        out_specs=pl.BlockSpec((1, BB, _C), lambda b, t: (t, b, 0)),
        out_shape=jax.ShapeDtypeStruct((_T - 1, _B, _C), jnp.float32),
        scratch_shapes=[pltpu.VMEM((BB, 1), bf)],
        compiler_params=pltpu.CompilerParams(
            dimension_semantics=("parallel", "arbitrary"),
        ),
        name="walk_loop",
    )(xT, base, eW1[:_DIN].astype(bf), eW2.astype(bf), eb2.reshape(1, -1))
    return jnp.swapaxes(outT, 0, 1)


# speedup vs baseline: 1.5734x; 1.0894x over previous
import jax
import jax.numpy as jnp
from jax.experimental import pallas as pl
from jax.experimental.pallas import tpu as pltpu

_B, _T, _DIN, _H, _C = 1024, 85, 20, 1024, 27
_OBJ = _T * 36


def _base_kernel(o_ref, oW1_ref, ob1_ref, oW2_ref, ob2_ref, eW1_ref,
                 eb1_ref, base_ref):
    h = jnp.dot(o_ref[...], oW1_ref[...],
                preferred_element_type=jnp.float32) + ob1_ref[...]
    h = jnp.maximum(h, 0.0)
    obj = jnp.dot(h, oW2_ref[...],
                  preferred_element_type=jnp.float32) + ob2_ref[...]
    objpad = jnp.concatenate(
        [jnp.zeros((obj.shape[0], _DIN), jnp.float32), obj], axis=1)
    base_ref[...] = jnp.dot(objpad, eW1_ref[...],
                            preferred_element_type=jnp.float32) + eb1_ref[...]


def _loop_kernel(xT_ref, base_ref, eW1_ref, w2_ref, eb2_ref, out_ref,
                 prev_ref):
    t = pl.program_id(1)
    xt = xT_ref[0].astype(jnp.bfloat16)

    @pl.when(t == 0)
    def _():
        prev_ref[...] = xt[:, 0:1]

    cur = jnp.concatenate([prev_ref[...], xt[:, 1:]], axis=1)
    w1 = eW1_ref[0:_DIN, :].astype(jnp.bfloat16)
    h = jnp.dot(cur, w1,
                preferred_element_type=jnp.float32) + base_ref[...]
    h = jnp.maximum(h, 0.0).astype(jnp.bfloat16)
    ew = jnp.dot(h, w2_ref[...].astype(jnp.bfloat16),
                 preferred_element_type=jnp.float32) + eb2_ref[...]
    prev_ref[...] = ew[:, 0:1].astype(jnp.bfloat16)
    out_ref[0] = ew


def kernel(o, x, oW1, ob1, oW2, ob2, eW1, eb1, eW2, eb2):
    MB = 256
    base = pl.pallas_call(
        _base_kernel,
        grid=(_B // MB,),
        in_specs=[
            pl.BlockSpec((MB, _OBJ), lambda i: (i, 0)),
            pl.BlockSpec((_OBJ, _H), lambda i: (0, 0)),
            pl.BlockSpec((1, _H), lambda i: (0, 0)),
            pl.BlockSpec((_H, _H), lambda i: (0, 0)),
            pl.BlockSpec((1, _H), lambda i: (0, 0)),
            pl.BlockSpec((_DIN + _H, _H), lambda i: (0, 0)),
            pl.BlockSpec((1, _H), lambda i: (0, 0)),
        ],
        out_specs=pl.BlockSpec((MB, _H), lambda i: (i, 0)),
        out_shape=jax.ShapeDtypeStruct((_B, _H), jnp.float32),
        compiler_params=pltpu.CompilerParams(
            dimension_semantics=("parallel",),
            vmem_limit_bytes=56 * 1024 * 1024,
        ),
        name="walk_base",
    )(o, oW1, ob1.reshape(1, -1), oW2, ob2.reshape(1, -1), eW1,
      eb1.reshape(1, -1))

    xT = jnp.swapaxes(x, 0, 1)
    BB = 512
    outT = pl.pallas_call(
        _loop_kernel,
        grid=(_B // BB, _T - 1),
        in_specs=[
            pl.BlockSpec((1, BB, _DIN), lambda b, t: (t, b, 0)),
            pl.BlockSpec((BB, _H), lambda b, t: (b, 0)),
            pl.BlockSpec((_DIN + _H, _H), lambda b, t: (0, 0)),
            pl.BlockSpec((_H, _C), lambda b, t: (0, 0)),
            pl.BlockSpec((1, _C), lambda b, t: (0, 0)),
        ],
        out_specs=pl.BlockSpec((1, BB, _C), lambda b, t: (t, b, 0)),
        out_shape=jax.ShapeDtypeStruct((_T - 1, _B, _C), jnp.float32),
        scratch_shapes=[pltpu.VMEM((BB, 1), jnp.bfloat16)],
        compiler_params=pltpu.CompilerParams(
            dimension_semantics=("parallel", "arbitrary"),
        ),
        name="walk_loop",
    )(xT, base, eW1, eW2, eb2.reshape(1, -1))
    return jnp.swapaxes(outT, 0, 1)


# speedup vs baseline: 1.9549x; 1.2425x over previous
import jax
import jax.numpy as jnp
from jax.experimental import pallas as pl
from jax.experimental.pallas import tpu as pltpu

_B, _T, _DIN, _H, _C = 1024, 85, 20, 1024, 27
_OBJ = _T * 36


def _base_kernel(o_ref, oW1_ref, ob1_ref, oW2_ref, ob2_ref, eW1_ref,
                 eb1_ref, base_ref):
    h = jnp.dot(o_ref[...], oW1_ref[...],
                preferred_element_type=jnp.float32) + ob1_ref[...]
    h = jnp.maximum(h, 0.0)
    obj = jnp.dot(h, oW2_ref[...],
                  preferred_element_type=jnp.float32) + ob2_ref[...]
    objpad = jnp.concatenate(
        [jnp.zeros((obj.shape[0], _DIN), jnp.float32), obj], axis=1)
    base_ref[...] = jnp.dot(objpad, eW1_ref[...],
                            preferred_element_type=jnp.float32) + eb1_ref[...]


_UNROLL = 6


def _loop_kernel(xT_ref, base_ref, eW1_ref, w2_ref, eb2_ref, out_ref,
                 prev_ref):
    c = pl.program_id(1)

    @pl.when(c == 0)
    def _():
        prev_ref[...] = xT_ref[0][:, 0:1].astype(jnp.bfloat16)

    prev = prev_ref[...]
    w1 = eW1_ref[0:_DIN, :].astype(jnp.bfloat16)
    w2 = w2_ref[...].astype(jnp.bfloat16)
    base = base_ref[...]
    for s in range(_UNROLL):
        xt = xT_ref[s].astype(jnp.bfloat16)
        cur = jnp.concatenate([prev, xt[:, 1:]], axis=1)
        h = jnp.dot(cur, w1, preferred_element_type=jnp.float32) + base
        h = jnp.maximum(h, 0.0).astype(jnp.bfloat16)
        ew = jnp.dot(h, w2,
                     preferred_element_type=jnp.float32) + eb2_ref[...]
        out_ref[s] = ew
        prev = ew[:, 0:1].astype(jnp.bfloat16)
    prev_ref[...] = prev


def kernel(o, x, oW1, ob1, oW2, ob2, eW1, eb1, eW2, eb2):
    MB = 256
    base = pl.pallas_call(
        _base_kernel,
        grid=(_B // MB,),
        in_specs=[
            pl.BlockSpec((MB, _OBJ), lambda i: (i, 0)),
            pl.BlockSpec((_OBJ, _H), lambda i: (0, 0)),
            pl.BlockSpec((1, _H), lambda i: (0, 0)),
            pl.BlockSpec((_H, _H), lambda i: (0, 0)),
            pl.BlockSpec((1, _H), lambda i: (0, 0)),
            pl.BlockSpec((_DIN + _H, _H), lambda i: (0, 0)),
            pl.BlockSpec((1, _H), lambda i: (0, 0)),
        ],
        out_specs=pl.BlockSpec((MB, _H), lambda i: (i, 0)),
        out_shape=jax.ShapeDtypeStruct((_B, _H), jnp.float32),
        compiler_params=pltpu.CompilerParams(
            dimension_semantics=("parallel",),
            vmem_limit_bytes=56 * 1024 * 1024,
        ),
        name="walk_base",
    )(o, oW1, ob1.reshape(1, -1), oW2, ob2.reshape(1, -1), eW1,
      eb1.reshape(1, -1))

    xT = jnp.swapaxes(x, 0, 1)
    BB = 512
    outT = pl.pallas_call(
        _loop_kernel,
        grid=(_B // BB, (_T - 1) // _UNROLL),
        in_specs=[
            pl.BlockSpec((_UNROLL, BB, _DIN), lambda b, t: (t, b, 0)),
            pl.BlockSpec((BB, _H), lambda b, t: (b, 0)),
            pl.BlockSpec((_DIN + _H, _H), lambda b, t: (0, 0)),
            pl.BlockSpec((_H, _C), lambda b, t: (0, 0)),
            pl.BlockSpec((1, _C), lambda b, t: (0, 0)),
        ],
        out_specs=pl.BlockSpec((_UNROLL, BB, _C), lambda b, t: (t, b, 0)),
        out_shape=jax.ShapeDtypeStruct((_T - 1, _B, _C), jnp.float32),
        scratch_shapes=[pltpu.VMEM((BB, 1), jnp.bfloat16)],
        compiler_params=pltpu.CompilerParams(
            dimension_semantics=("parallel", "arbitrary"),
        ),
        name="walk_loop",
    )(xT, base, eW1, eW2, eb2.reshape(1, -1))
    return jnp.swapaxes(outT, 0, 1)
